# baseline (speedup 1.0000x reference)
import functools

import jax
import jax.numpy as jnp
from jax import lax
from jax.experimental import pallas as pl
from jax.experimental.pallas import tpu as pltpu
from jax.experimental.pallas import tpu_sc as plsc

_VOCAB = 1000000
_D = 64
_H = 256
_B = 4096
_SEQ = 200

_NC = 2
_NS = 16
_NW = _NC * _NS
_BPW = _B // _NW

_SEQ_A = 104
_SEQ_B = _SEQ - _SEQ_A
_NV = _D // 16


_UNROLL = 8


def _pool_body(wi_hbm, tbl_hbm, out_hbm, idx_v, rows_a0, rows_b0, rows_a1,
               rows_b1, out_v, sem_a0, sem_b0, sem_a1, sem_b1):
  c = lax.axis_index("c")
  s = lax.axis_index("s")
  wid = s * _NC + c
  base = wid * _BPW

  pltpu.sync_copy(wi_hbm.at[pl.ds(base, _BPW)], idx_v)

  def issue(r, rows_a, rows_b, sem_a, sem_b):
    pltpu.async_copy(
        tbl_hbm.at[idx_v.at[r, pl.ds(0, _SEQ_A)]], rows_a, sem_a)
    pltpu.async_copy(
        tbl_hbm.at[idx_v.at[r, pl.ds(_SEQ_A, _SEQ_B)]], rows_b, sem_b)

  def wait(rows_a, rows_b, sem_a, sem_b):
    pltpu.make_async_copy(tbl_hbm.at[pl.ds(0, _SEQ_A)], rows_a, sem_a).wait()
    pltpu.make_async_copy(tbl_hbm.at[pl.ds(0, _SEQ_B)], rows_b, sem_b).wait()

  def reduce_rows(rows_ref, n, acc):
    def jbody(j, acc):
      for k in range(_UNROLL):
        acc = tuple(
            acc[d] + rows_ref[j * _UNROLL + k, pl.ds(d * 16, 16)]
            for d in range(_NV))
      return acc
    return lax.fori_loop(0, n // _UNROLL, jbody, acc)

  def consume(r, rows_a, rows_b, sem_a, sem_b):
    wait(rows_a, rows_b, sem_a, sem_b)
    zero = tuple(jnp.zeros((16,), jnp.float32) for _ in range(_NV))
    acc = reduce_rows(rows_a, _SEQ_A, zero)
    acc = reduce_rows(rows_b, _SEQ_B, acc)
    for d in range(_NV):
      out_v[r, pl.ds(d * 16, 16)] = acc[d]

  issue(0, rows_a0, rows_b0, sem_a0, sem_b0)

  def pair_body(p, carry):
    r0 = 2 * p
    issue(r0 + 1, rows_a1, rows_b1, sem_a1, sem_b1)
    consume(r0, rows_a0, rows_b0, sem_a0, sem_b0)
    issue(jnp.minimum(r0 + 2, _BPW - 1), rows_a0, rows_b0, sem_a0, sem_b0)
    consume(r0 + 1, rows_a1, rows_b1, sem_a1, sem_b1)
    return carry

  lax.fori_loop(0, _BPW // 2, pair_body, 0)
  wait(rows_a0, rows_b0, sem_a0, sem_b0)
  pltpu.sync_copy(out_v, out_hbm.at[pl.ds(base, _BPW)])


@jax.jit
def _pool(word_indices, table):
  mesh = plsc.VectorSubcoreMesh(
      core_axis_name="c", subcore_axis_name="s",
      num_cores=_NC, num_subcores=_NS)
  return pl.kernel(
      _pool_body,
      out_type=jax.ShapeDtypeStruct((_B, _D), jnp.float32),
      mesh=mesh,
      compiler_params=pltpu.CompilerParams(use_tc_tiling_on_sc=False),
      scratch_types=[
          pltpu.VMEM((_BPW, _SEQ), jnp.int32),
          pltpu.VMEM((_SEQ_A, _D), jnp.float32),
          pltpu.VMEM((_SEQ_B, _D), jnp.float32),
          pltpu.VMEM((_SEQ_A, _D), jnp.float32),
          pltpu.VMEM((_SEQ_B, _D), jnp.float32),
          pltpu.VMEM((_BPW, _D), jnp.float32),
          pltpu.SemaphoreType.DMA,
          pltpu.SemaphoreType.DMA,
          pltpu.SemaphoreType.DMA,
          pltpu.SemaphoreType.DMA,
      ],
  )(word_indices, table)


_CB = 1024


def _tr_kernel(xT_ref, o_ref):
  x = xT_ref[...].reshape(_D, _CB // 2, 2)
  o_ref[...] = jnp.transpose(x, (1, 2, 0)).reshape(_CB // 2, 2 * _D)


@jax.jit
def _relayout(tableT):
  return pl.pallas_call(
      _tr_kernel,
      grid=((_VOCAB + _CB - 1) // _CB,),
      in_specs=[pl.BlockSpec((_D, _CB), lambda g: (0, g))],
      out_specs=pl.BlockSpec((_CB // 2, 128), lambda g: (g, 0)),
      out_shape=jax.ShapeDtypeStruct((_VOCAB // 2, 128), jnp.float32),
  )(tableT)


def _mlp_kernel(x_ref, w1_ref, b1_ref, w2_ref, b2_ref, o_ref):
  x = x_ref[...] * (1.0 / _SEQ)
  h = jnp.dot(x, w1_ref[...], preferred_element_type=jnp.float32)
  h = jnp.maximum(h + b1_ref[...], 0.0)
  logits = jnp.dot(h, w2_ref[...], preferred_element_type=jnp.float32)
  logits = logits + b2_ref[...]
  m = jnp.max(logits, axis=1, keepdims=True)
  e = jnp.exp(logits - m)
  o_ref[...] = e / jnp.sum(e, axis=1, keepdims=True)


@jax.jit
def _mlp(sums, W1, b1, W2p, b2p):
  return pl.pallas_call(
      _mlp_kernel,
      out_shape=jax.ShapeDtypeStruct((_B, 128), jnp.float32),
  )(sums, W1, b1, W2p, b2p)


def kernel(word_indices, table, W1, b1, W2, b2):
  table_lin = jnp.reshape(_relayout(table.T), (_VOCAB, _D))
  sums = _pool(word_indices, table_lin)
  W2p = jnp.pad(W2, ((0, 0), (0, 128 - W2.shape[1])))
  b2p = jnp.concatenate(
      [b2, jnp.full((128 - b2.shape[0],), -1e30, jnp.float32)])
  out = _mlp(sums, W1, b1.reshape(1, _H), W2p, b2p.reshape(1, 128))
  return out[:, :2]


# speedup vs baseline: 24.2845x; 24.2845x over previous
import functools

import jax
import jax.numpy as jnp
from jax import lax
from jax.experimental import pallas as pl
from jax.experimental.pallas import tpu as pltpu
from jax.experimental.pallas import tpu_sc as plsc

_VOCAB = 1000000
_D = 64
_H = 256
_B = 4096
_SEQ = 200

_NC = 2
_NS = 16
_NW = _NC * _NS
_BPW = _B // _NW

_SEQ_A = 104
_SEQ_B = _SEQ - _SEQ_A
_NV = _D // 16


_UNROLL = 8


def _pool_body(wi_hbm, tbl_hbm, out_hbm, idx_v, rows_a0, rows_b0, rows_a1,
               rows_b1, out_v, sem_a0, sem_b0, sem_a1, sem_b1):
  c = lax.axis_index("c")
  s = lax.axis_index("s")
  wid = s * _NC + c
  base = wid * _BPW

  pltpu.sync_copy(wi_hbm.at[pl.ds(base, _BPW)], idx_v)

  def issue(r, rows_a, rows_b, sem_a, sem_b):
    pltpu.async_copy(
        tbl_hbm.at[idx_v.at[r, pl.ds(0, _SEQ_A)]], rows_a, sem_a)
    pltpu.async_copy(
        tbl_hbm.at[idx_v.at[r, pl.ds(_SEQ_A, _SEQ_B)]], rows_b, sem_b)

  def wait(rows_a, rows_b, sem_a, sem_b):
    pltpu.make_async_copy(tbl_hbm.at[pl.ds(0, _SEQ_A)], rows_a, sem_a).wait()
    pltpu.make_async_copy(tbl_hbm.at[pl.ds(0, _SEQ_B)], rows_b, sem_b).wait()

  def reduce_rows(rows_ref, n, acc):
    def jbody(j, acc):
      for k in range(_UNROLL):
        acc = tuple(
            acc[d] + rows_ref[j * _UNROLL + k, pl.ds(d * 16, 16)]
            for d in range(_NV))
      return acc
    return lax.fori_loop(0, n // _UNROLL, jbody, acc)

  def consume(r, rows_a, rows_b, sem_a, sem_b):
    wait(rows_a, rows_b, sem_a, sem_b)
    zero = tuple(jnp.zeros((16,), jnp.float32) for _ in range(_NV))
    acc = reduce_rows(rows_a, _SEQ_A, zero)
    acc = reduce_rows(rows_b, _SEQ_B, acc)
    for d in range(_NV):
      out_v[r, pl.ds(d * 16, 16)] = acc[d]

  issue(0, rows_a0, rows_b0, sem_a0, sem_b0)

  def pair_body(p, carry):
    r0 = 2 * p
    issue(r0 + 1, rows_a1, rows_b1, sem_a1, sem_b1)
    consume(r0, rows_a0, rows_b0, sem_a0, sem_b0)
    issue(jnp.minimum(r0 + 2, _BPW - 1), rows_a0, rows_b0, sem_a0, sem_b0)
    consume(r0 + 1, rows_a1, rows_b1, sem_a1, sem_b1)
    return carry

  lax.fori_loop(0, _BPW // 2, pair_body, 0)
  wait(rows_a0, rows_b0, sem_a0, sem_b0)
  pltpu.sync_copy(out_v, out_hbm.at[pl.ds(base, _BPW)])


@jax.jit
def _pool(word_indices, table):
  mesh = plsc.VectorSubcoreMesh(
      core_axis_name="c", subcore_axis_name="s",
      num_cores=_NC, num_subcores=_NS)
  return pl.kernel(
      _pool_body,
      out_type=jax.ShapeDtypeStruct((_B, _D), jnp.float32),
      mesh=mesh,
      compiler_params=pltpu.CompilerParams(use_tc_tiling_on_sc=False),
      scratch_types=[
          pltpu.VMEM((_BPW, _SEQ), jnp.int32),
          pltpu.VMEM((_SEQ_A, _D), jnp.float32),
          pltpu.VMEM((_SEQ_B, _D), jnp.float32),
          pltpu.VMEM((_SEQ_A, _D), jnp.float32),
          pltpu.VMEM((_SEQ_B, _D), jnp.float32),
          pltpu.VMEM((_BPW, _D), jnp.float32),
          pltpu.SemaphoreType.DMA,
          pltpu.SemaphoreType.DMA,
          pltpu.SemaphoreType.DMA,
          pltpu.SemaphoreType.DMA,
      ],
  )(word_indices, table)


_CBH = 8192
_NB = 62
_S = _NB * _CBH
_LASTB = (_VOCAB - 1) // _CBH


def _tr_kernel(lo_ref, hi_ref, o_ref):
  o_ref[...] = jnp.concatenate(
      [jnp.transpose(lo_ref[...]), jnp.transpose(hi_ref[...])], axis=1)


@jax.jit
def _relayout(tableT):
  return pl.pallas_call(
      _tr_kernel,
      grid=(_NB,),
      in_specs=[
          pl.BlockSpec((_D, _CBH), lambda g: (0, g)),
          pl.BlockSpec((_D, _CBH), lambda g: (0, jnp.minimum(g + _NB, _LASTB))),
      ],
      out_specs=pl.BlockSpec((_CBH, 128), lambda g: (g, 0)),
      out_shape=jax.ShapeDtypeStruct((_S, 128), jnp.float32),
  )(tableT, tableT)


def _mlp_kernel(x_ref, w1_ref, b1_ref, w2_ref, b2_ref, o_ref):
  x = x_ref[...] * (1.0 / _SEQ)
  h = jnp.dot(x, w1_ref[...], preferred_element_type=jnp.float32)
  h = jnp.maximum(h + b1_ref[...], 0.0)
  logits = jnp.dot(h, w2_ref[...], preferred_element_type=jnp.float32)
  logits = logits + b2_ref[...]
  m = jnp.max(logits, axis=1, keepdims=True)
  e = jnp.exp(logits - m)
  o_ref[...] = e / jnp.sum(e, axis=1, keepdims=True)


@jax.jit
def _mlp(sums, W1, b1, W2p, b2p):
  return pl.pallas_call(
      _mlp_kernel,
      out_shape=jax.ShapeDtypeStruct((_B, 128), jnp.float32),
  )(sums, W1, b1, W2p, b2p)


def kernel(word_indices, table, W1, b1, W2, b2):
  table_lin = jnp.reshape(_relayout(table.T), (2 * _S, _D))
  wi = jnp.where(word_indices < _S, word_indices * 2,
                 (word_indices - _S) * 2 + 1)
  sums = _pool(wi, table_lin)
  W2p = jnp.pad(W2, ((0, 0), (0, 128 - W2.shape[1])))
  b2p = jnp.concatenate(
      [b2, jnp.full((128 - b2.shape[0],), -1e30, jnp.float32)])
  out = _mlp(sums, W1, b1.reshape(1, _H), W2p, b2p.reshape(1, 128))
  return out[:, :2]


# speedup vs baseline: 27.8326x; 1.1461x over previous
import functools

import jax
import jax.numpy as jnp
from jax import lax
from jax.experimental import pallas as pl
from jax.experimental.pallas import tpu as pltpu
from jax.experimental.pallas import tpu_sc as plsc

_VOCAB = 1000000
_D = 64
_H = 256
_B = 4096
_SEQ = 200

_NC = 2
_NS = 16
_NW = _NC * _NS
_BPW = _B // _NW

_SEQ_A = 104
_SEQ_B = _SEQ - _SEQ_A
_NV = _D // 16


_UNROLL = 8
_NBUF = 4


def _pool_body(wi_hbm, tbl_hbm, out_hbm, idx_v, out_v, rows, sems):
  c = lax.axis_index("c")
  s = lax.axis_index("s")
  wid = s * _NC + c
  base = wid * _BPW

  pltpu.sync_copy(wi_hbm.at[pl.ds(base, _BPW)], idx_v)

  def issue(r, t):
    rows_a, rows_b = rows[t]
    sem_a, sem_b = sems[t]
    pltpu.async_copy(
        tbl_hbm.at[idx_v.at[r, pl.ds(0, _SEQ_A)]], rows_a, sem_a)
    pltpu.async_copy(
        tbl_hbm.at[idx_v.at[r, pl.ds(_SEQ_A, _SEQ_B)]], rows_b, sem_b)

  def wait(t):
    rows_a, rows_b = rows[t]
    sem_a, sem_b = sems[t]
    pltpu.make_async_copy(tbl_hbm.at[pl.ds(0, _SEQ_A)], rows_a, sem_a).wait()
    pltpu.make_async_copy(tbl_hbm.at[pl.ds(0, _SEQ_B)], rows_b, sem_b).wait()

  def reduce_rows(rows_ref, n, acc):
    def jbody(j, acc):
      for k in range(_UNROLL):
        acc = tuple(
            acc[d] + rows_ref[j * _UNROLL + k, pl.ds(d * 16, 16)]
            for d in range(_NV))
      return acc
    return lax.fori_loop(0, n // _UNROLL, jbody, acc)

  def consume(r, t):
    wait(t)
    rows_a, rows_b = rows[t]
    zero = tuple(jnp.zeros((16,), jnp.float32) for _ in range(_NV))
    acc = reduce_rows(rows_a, _SEQ_A, zero)
    acc = reduce_rows(rows_b, _SEQ_B, acc)
    for d in range(_NV):
      out_v[r, pl.ds(d * 16, 16)] = acc[d]

  for t in range(_NBUF - 1):
    issue(t, t)

  def ring_body(q, carry):
    r0 = _NBUF * q
    for t in range(_NBUF):
      issue(jnp.minimum(r0 + t + _NBUF - 1, _BPW - 1), (t + _NBUF - 1) % _NBUF)
      consume(r0 + t, t)
    return carry

  lax.fori_loop(0, _BPW // _NBUF, ring_body, 0)
  for t in range(_NBUF - 1):
    wait(t)
  pltpu.sync_copy(out_v, out_hbm.at[pl.ds(base, _BPW)])


@jax.jit
def _pool(word_indices, table):
  mesh = plsc.VectorSubcoreMesh(
      core_axis_name="c", subcore_axis_name="s",
      num_cores=_NC, num_subcores=_NS)
  return pl.kernel(
      _pool_body,
      out_type=jax.ShapeDtypeStruct((_B, _D), jnp.float32),
      mesh=mesh,
      compiler_params=pltpu.CompilerParams(use_tc_tiling_on_sc=False),
      scratch_types=[
          pltpu.VMEM((_BPW, _SEQ), jnp.int32),
          pltpu.VMEM((_BPW, _D), jnp.float32),
          [(pltpu.VMEM((_SEQ_A, _D), jnp.float32),
            pltpu.VMEM((_SEQ_B, _D), jnp.float32)) for _ in range(_NBUF)],
          [(pltpu.SemaphoreType.DMA, pltpu.SemaphoreType.DMA)
           for _ in range(_NBUF)],
      ],
  )(word_indices, table)


_CBH = 16384
_NB = 31
_S = _NB * _CBH
_LASTB = (_VOCAB - 1) // _CBH


def _tr_kernel(lo_ref, hi_ref, o_ref):
  o_ref[...] = jnp.concatenate(
      [jnp.transpose(lo_ref[...]), jnp.transpose(hi_ref[...])], axis=1)


@jax.jit
def _relayout(tableT):
  return pl.pallas_call(
      _tr_kernel,
      grid=(_NB,),
      in_specs=[
          pl.BlockSpec((_D, _CBH), lambda g: (0, g)),
          pl.BlockSpec((_D, _CBH), lambda g: (0, jnp.minimum(g + _NB, _LASTB))),
      ],
      out_specs=pl.BlockSpec((_CBH, 128), lambda g: (g, 0)),
      out_shape=jax.ShapeDtypeStruct((_S, 128), jnp.float32),
  )(tableT, tableT)


def _mlp_kernel(x_ref, w1_ref, b1_ref, w2_ref, b2_ref, o_ref):
  x = x_ref[...] * (1.0 / _SEQ)
  h = jnp.dot(x, w1_ref[...], preferred_element_type=jnp.float32)
  h = jnp.maximum(h + b1_ref[...], 0.0)
  logits = jnp.dot(h, w2_ref[...], preferred_element_type=jnp.float32)
  logits = logits + b2_ref[...]
  m = jnp.max(logits, axis=1, keepdims=True)
  e = jnp.exp(logits - m)
  o_ref[...] = e / jnp.sum(e, axis=1, keepdims=True)


@jax.jit
def _mlp(sums, W1, b1, W2p, b2p):
  return pl.pallas_call(
      _mlp_kernel,
      out_shape=jax.ShapeDtypeStruct((_B, 128), jnp.float32),
  )(sums, W1, b1, W2p, b2p)


def kernel(word_indices, table, W1, b1, W2, b2):
  table_lin = jnp.reshape(_relayout(table.T), (2 * _S, _D))
  wi = jnp.where(word_indices < _S, word_indices * 2,
                 (word_indices - _S) * 2 + 1)
  sums = _pool(wi, table_lin)
  W2p = jnp.pad(W2, ((0, 0), (0, 128 - W2.shape[1])))
  b2p = jnp.concatenate(
      [b2, jnp.full((128 - b2.shape[0],), -1e30, jnp.float32)])
  out = _mlp(sums, W1, b1.reshape(1, _H), W2p, b2p.reshape(1, 128))
  return out[:, :2]


# speedup vs baseline: 32.7974x; 1.1784x over previous
import functools

import jax
import jax.numpy as jnp
from jax import lax
from jax.experimental import pallas as pl
from jax.experimental.pallas import tpu as pltpu
from jax.experimental.pallas import tpu_sc as plsc

_VOCAB = 1000000
_D = 64
_H = 256
_B = 4096
_SEQ = 200

_NC = 2
_NS = 16
_NW = _NC * _NS
_BPW = _B // _NW

_SEQ_A = 104
_SEQ_B = _SEQ - _SEQ_A
_NV = _D // 16

_CBH = 8192
_NBK = 32
_S4 = _NBK * _CBH
_LASTB = (_VOCAB - 1) // _CBH


def _pack(a, b):
  ua = lax.convert_element_type(
      lax.bitcast_convert_type(a.astype(jnp.bfloat16), jnp.uint16),
      jnp.uint32)
  ub = lax.convert_element_type(
      lax.bitcast_convert_type(b.astype(jnp.bfloat16), jnp.uint16),
      jnp.uint32)
  return lax.bitcast_convert_type(ua | (ub << jnp.uint32(16)), jnp.float32)


def _tr_kernel(p0_ref, p1_ref, p2_ref, p3_ref, o_ref):
  t1 = jnp.transpose(jnp.concatenate([p0_ref[...], p1_ref[...]], axis=0))
  t2 = jnp.transpose(jnp.concatenate([p2_ref[...], p3_ref[...]], axis=0))
  o_ref[...] = jnp.concatenate([
      _pack(t1[:, 0:32], t1[:, 32:64]),
      _pack(t1[:, 64:96], t1[:, 96:128]),
      _pack(t2[:, 0:32], t2[:, 32:64]),
      _pack(t2[:, 64:96], t2[:, 96:128]),
  ], axis=1)


@jax.jit
def _relayout(tableT):
  return pl.pallas_call(
      _tr_kernel,
      grid=(_NBK,),
      in_specs=[
          pl.BlockSpec((_D, _CBH), lambda g: (0, g)),
          pl.BlockSpec((_D, _CBH), lambda g: (0, g + _NBK)),
          pl.BlockSpec((_D, _CBH), lambda g: (0, g + 2 * _NBK)),
          pl.BlockSpec((_D, _CBH),
                       lambda g: (0, jnp.minimum(g + 3 * _NBK, _LASTB))),
      ],
      out_specs=pl.BlockSpec((_CBH, 128), lambda g: (g, 0)),
      out_shape=jax.ShapeDtypeStruct((_S4, 128), jnp.float32),
  )(tableT, tableT, tableT, tableT)


_UNROLL = 8
_NBUF = 4
_PW = _D // 2


def _pool_body(wi_hbm, tbl_hbm, out_hbm, idx_v, out_v, rows, sems):
  c = lax.axis_index("c")
  s = lax.axis_index("s")
  wid = s * _NC + c
  base = wid * _BPW

  pltpu.sync_copy(wi_hbm.at[pl.ds(base, _BPW)], idx_v)

  def issue(r, t):
    rows_a, rows_b = rows[t]
    sem_a, sem_b = sems[t]
    pltpu.async_copy(
        tbl_hbm.at[idx_v.at[r, pl.ds(0, _SEQ_A)]], rows_a, sem_a)
    pltpu.async_copy(
        tbl_hbm.at[idx_v.at[r, pl.ds(_SEQ_A, _SEQ_B)]], rows_b, sem_b)

  def wait(t):
    rows_a, rows_b = rows[t]
    sem_a, sem_b = sems[t]
    pltpu.make_async_copy(tbl_hbm.at[pl.ds(0, _SEQ_A)], rows_a, sem_a).wait()
    pltpu.make_async_copy(tbl_hbm.at[pl.ds(0, _SEQ_B)], rows_b, sem_b).wait()

  def unpack_add(w, acc_lo, acc_hi):
    u = plsc.bitcast(w, jnp.uint32)
    lo = plsc.bitcast(lax.shift_left(u, jnp.uint32(16)), jnp.float32)
    hi = plsc.bitcast(u & jnp.uint32(0xFFFF0000), jnp.float32)
    return acc_lo + lo, acc_hi + hi

  def reduce_rows(rows_ref, n, acc):
    def jbody(j, acc):
      a_lo, b_lo, a_hi, b_hi = acc
      for k in range(_UNROLL):
        wa = rows_ref[j * _UNROLL + k, pl.ds(0, 16)]
        wb = rows_ref[j * _UNROLL + k, pl.ds(16, 16)]
        a_lo, a_hi = unpack_add(wa, a_lo, a_hi)
        b_lo, b_hi = unpack_add(wb, b_lo, b_hi)
      return a_lo, b_lo, a_hi, b_hi
    return lax.fori_loop(0, n // _UNROLL, jbody, acc)

  def consume(r, t):
    wait(t)
    rows_a, rows_b = rows[t]
    zero = tuple(jnp.zeros((16,), jnp.float32) for _ in range(4))
    acc = reduce_rows(rows_a, _SEQ_A, zero)
    acc = reduce_rows(rows_b, _SEQ_B, acc)
    for d in range(4):
      out_v[r, pl.ds(d * 16, 16)] = acc[d]

  for t in range(_NBUF - 1):
    issue(t, t)

  def ring_body(q, carry):
    r0 = _NBUF * q
    for t in range(_NBUF):
      issue(jnp.minimum(r0 + t + _NBUF - 1, _BPW - 1), (t + _NBUF - 1) % _NBUF)
      consume(r0 + t, t)
    return carry

  lax.fori_loop(0, _BPW // _NBUF, ring_body, 0)
  for t in range(_NBUF - 1):
    wait(t)
  pltpu.sync_copy(out_v, out_hbm.at[pl.ds(base, _BPW)])


@jax.jit
def _pool(word_indices, table_pk):
  mesh = plsc.VectorSubcoreMesh(
      core_axis_name="c", subcore_axis_name="s",
      num_cores=_NC, num_subcores=_NS)
  return pl.kernel(
      _pool_body,
      out_type=jax.ShapeDtypeStruct((_B, _D), jnp.float32),
      mesh=mesh,
      compiler_params=pltpu.CompilerParams(
          use_tc_tiling_on_sc=False, needs_layout_passes=False),
      scratch_types=[
          pltpu.VMEM((_BPW, _SEQ), jnp.int32),
          pltpu.VMEM((_BPW, _D), jnp.float32),
          [(pltpu.VMEM((_SEQ_A, _PW), jnp.float32),
            pltpu.VMEM((_SEQ_B, _PW), jnp.float32)) for _ in range(_NBUF)],
          [(pltpu.SemaphoreType.DMA, pltpu.SemaphoreType.DMA)
           for _ in range(_NBUF)],
      ],
  )(word_indices, table_pk)


def _mlp_kernel(x_ref, w1_ref, b1_ref, w2_ref, b2_ref, o_ref):
  x = x_ref[...] * (1.0 / _SEQ)
  h = jnp.dot(x, w1_ref[...], preferred_element_type=jnp.float32)
  h = jnp.maximum(h + b1_ref[...], 0.0)
  logits = jnp.dot(h, w2_ref[...], preferred_element_type=jnp.float32)
  logits = logits + b2_ref[...]
  m = jnp.max(logits, axis=1, keepdims=True)
  e = jnp.exp(logits - m)
  o_ref[...] = e / jnp.sum(e, axis=1, keepdims=True)


@jax.jit
def _mlp(sums, W1, b1, W2p, b2p):
  return pl.pallas_call(
      _mlp_kernel,
      out_shape=jax.ShapeDtypeStruct((_B, 128), jnp.float32),
  )(sums, W1, b1, W2p, b2p)


def kernel(word_indices, table, W1, b1, W2, b2):
  table_pk = jnp.reshape(_relayout(table.T), (4 * _S4, _PW))
  wi = 4 * (word_indices % _S4) + word_indices // _S4
  sums = _pool(wi, table_pk)
  W2p = jnp.pad(W2, ((0, 0), (0, 128 - W2.shape[1])))
  b2p = jnp.concatenate(
      [b2, jnp.full((128 - b2.shape[0],), -1e30, jnp.float32)])
  out = _mlp(sums, W1, b1.reshape(1, _H), W2p, b2p.reshape(1, 128))
  return out[:, :2]


# speedup vs baseline: 41.8159x; 1.2750x over previous
import functools

import jax
import jax.numpy as jnp
from jax import lax
from jax.experimental import pallas as pl
from jax.experimental.pallas import tpu as pltpu
from jax.experimental.pallas import tpu_sc as plsc

_VOCAB = 1000000
_D = 64
_H = 256
_B = 4096
_SEQ = 200

_NC = 2
_NS = 16
_NW = _NC * _NS
_BPW = _B // _NW

_SEQ_A = 104
_SEQ_B = _SEQ - _SEQ_A
_NV = _D // 16

_CBH = 8192
_NBK = 32
_S4 = _NBK * _CBH
_LASTB = (_VOCAB - 1) // _CBH


def _rne16(u):
  one = jnp.uint32(1)
  return u + ((u >> jnp.uint32(16)) & one) + jnp.uint32(0x7FFF)


def _pack(a, b):
  ua = _rne16(lax.bitcast_convert_type(a, jnp.uint32)) >> jnp.uint32(16)
  ub = _rne16(lax.bitcast_convert_type(b, jnp.uint32)) & jnp.uint32(0xFFFF0000)
  return lax.bitcast_convert_type(ua | ub, jnp.float32)


def _tr_kernel(p0_ref, p1_ref, p2_ref, p3_ref, o_ref):
  parts = []
  for pk_ref in (p0_ref, p1_ref, p2_ref, p3_ref):
    x = pk_ref[...]
    parts.append(_pack(x[0:32, :], x[32:64, :]))
  o_ref[...] = jnp.transpose(jnp.concatenate(parts, axis=0))


@jax.jit
def _relayout(tableT):
  return pl.pallas_call(
      _tr_kernel,
      grid=(_NBK,),
      in_specs=[
          pl.BlockSpec((_D, _CBH), lambda g: (0, g)),
          pl.BlockSpec((_D, _CBH), lambda g: (0, g + _NBK)),
          pl.BlockSpec((_D, _CBH), lambda g: (0, g + 2 * _NBK)),
          pl.BlockSpec((_D, _CBH),
                       lambda g: (0, jnp.minimum(g + 3 * _NBK, _LASTB))),
      ],
      out_specs=pl.BlockSpec((_CBH, 128), lambda g: (g, 0)),
      out_shape=jax.ShapeDtypeStruct((_S4, 128), jnp.float32),
  )(tableT, tableT, tableT, tableT)


_UNROLL = 8
_NBUF = 4
_PW = _D // 2


def _pool_body(wi_hbm, tbl_hbm, out_hbm, idx_v, out_v, rows, sems):
  c = lax.axis_index("c")
  s = lax.axis_index("s")
  wid = s * _NC + c
  base = wid * _BPW

  pltpu.sync_copy(wi_hbm.at[pl.ds(base, _BPW)], idx_v)

  def issue(r, t):
    rows_a, rows_b = rows[t]
    sem_a, sem_b = sems[t]
    pltpu.async_copy(
        tbl_hbm.at[idx_v.at[r, pl.ds(0, _SEQ_A)]], rows_a, sem_a)
    pltpu.async_copy(
        tbl_hbm.at[idx_v.at[r, pl.ds(_SEQ_A, _SEQ_B)]], rows_b, sem_b)

  def wait(t):
    rows_a, rows_b = rows[t]
    sem_a, sem_b = sems[t]
    pltpu.make_async_copy(tbl_hbm.at[pl.ds(0, _SEQ_A)], rows_a, sem_a).wait()
    pltpu.make_async_copy(tbl_hbm.at[pl.ds(0, _SEQ_B)], rows_b, sem_b).wait()

  def unpack_add(w, acc_lo, acc_hi):
    u = plsc.bitcast(w, jnp.uint32)
    lo = plsc.bitcast(lax.shift_left(u, jnp.uint32(16)), jnp.float32)
    hi = plsc.bitcast(u & jnp.uint32(0xFFFF0000), jnp.float32)
    return acc_lo + lo, acc_hi + hi

  def reduce_rows(rows_ref, n, acc):
    def jbody(j, acc):
      a_lo, b_lo, a_hi, b_hi = acc
      for k in range(_UNROLL):
        wa = rows_ref[j * _UNROLL + k, pl.ds(0, 16)]
        wb = rows_ref[j * _UNROLL + k, pl.ds(16, 16)]
        a_lo, a_hi = unpack_add(wa, a_lo, a_hi)
        b_lo, b_hi = unpack_add(wb, b_lo, b_hi)
      return a_lo, b_lo, a_hi, b_hi
    return lax.fori_loop(0, n // _UNROLL, jbody, acc)

  def consume(r, t):
    wait(t)
    rows_a, rows_b = rows[t]
    zero = tuple(jnp.zeros((16,), jnp.float32) for _ in range(4))
    acc = reduce_rows(rows_a, _SEQ_A, zero)
    acc = reduce_rows(rows_b, _SEQ_B, acc)
    for d in range(4):
      out_v[r, pl.ds(d * 16, 16)] = acc[d]

  for t in range(_NBUF - 1):
    issue(t, t)

  def ring_body(q, carry):
    r0 = _NBUF * q
    for t in range(_NBUF):
      issue(jnp.minimum(r0 + t + _NBUF - 1, _BPW - 1), (t + _NBUF - 1) % _NBUF)
      consume(r0 + t, t)
    return carry

  lax.fori_loop(0, _BPW // _NBUF, ring_body, 0)
  for t in range(_NBUF - 1):
    wait(t)
  pltpu.sync_copy(out_v, out_hbm.at[pl.ds(base, _BPW)])


@jax.jit
def _pool(word_indices, table_pk):
  mesh = plsc.VectorSubcoreMesh(
      core_axis_name="c", subcore_axis_name="s",
      num_cores=_NC, num_subcores=_NS)
  return pl.kernel(
      _pool_body,
      out_type=jax.ShapeDtypeStruct((_B, _D), jnp.float32),
      mesh=mesh,
      compiler_params=pltpu.CompilerParams(
          use_tc_tiling_on_sc=False, needs_layout_passes=False),
      scratch_types=[
          pltpu.VMEM((_BPW, _SEQ), jnp.int32),
          pltpu.VMEM((_BPW, _D), jnp.float32),
          [(pltpu.VMEM((_SEQ_A, _PW), jnp.float32),
            pltpu.VMEM((_SEQ_B, _PW), jnp.float32)) for _ in range(_NBUF)],
          [(pltpu.SemaphoreType.DMA, pltpu.SemaphoreType.DMA)
           for _ in range(_NBUF)],
      ],
  )(word_indices, table_pk)


def _mlp_kernel(x_ref, w1_ref, b1_ref, w2_ref, b2_ref, o_ref):
  x = x_ref[...] * (1.0 / _SEQ)
  h = jnp.dot(x, w1_ref[...], preferred_element_type=jnp.float32)
  h = jnp.maximum(h + b1_ref[...], 0.0)
  logits = jnp.dot(h, w2_ref[...], preferred_element_type=jnp.float32)
  logits = logits + b2_ref[...]
  m = jnp.max(logits, axis=1, keepdims=True)
  e = jnp.exp(logits - m)
  o_ref[...] = e / jnp.sum(e, axis=1, keepdims=True)


@jax.jit
def _mlp(sums, W1, b1, W2p, b2p):
  return pl.pallas_call(
      _mlp_kernel,
      out_shape=jax.ShapeDtypeStruct((_B, 128), jnp.float32),
  )(sums, W1, b1, W2p, b2p)


def kernel(word_indices, table, W1, b1, W2, b2):
  table_pk = jnp.reshape(_relayout(table.T), (4 * _S4, _PW))
  wi = 4 * (word_indices % _S4) + word_indices // _S4
  sums = _pool(wi, table_pk)
  W2p = jnp.pad(W2, ((0, 0), (0, 128 - W2.shape[1])))
  b2p = jnp.concatenate(
      [b2, jnp.full((128 - b2.shape[0],), -1e30, jnp.float32)])
  out = _mlp(sums, W1, b1.reshape(1, _H), W2p, b2p.reshape(1, 128))
  return out[:, :2]
